# baseline (speedup 1.0000x reference)
import functools

import jax
import jax.numpy as jnp
from jax import lax
from jax.experimental import pallas as pl
from jax.experimental.pallas import tpu as pltpu
from jax.experimental.pallas import tpu_sc as plsc


_MESH = plsc.VectorSubcoreMesh(core_axis_name="c", subcore_axis_name="s")


@functools.partial(
    pl.kernel,
    out_type=(
        jax.ShapeDtypeStruct((2,), jnp.float32),
        jax.ShapeDtypeStruct((4,), jnp.float32),
    ),
    mesh=_MESH,
    scratch_types=[pltpu.VMEM((16,), jnp.float32)],
)
def _sc_add_one(a0_hbm, a1_hbm, o0_hbm, o1_hbm, v):
    cid = lax.axis_index("c")
    sid = lax.axis_index("s")

    @pl.when(jnp.logical_and(cid == 0, sid == 0))
    def _():
        pltpu.sync_copy(a0_hbm, v.at[pl.ds(0, 2)])
        v[...] = v[...] + 1.0
        pltpu.sync_copy(v.at[pl.ds(0, 2)], o0_hbm)

    @pl.when(jnp.logical_and(cid == 1, sid == 0))
    def _():
        pltpu.sync_copy(a1_hbm, v.at[pl.ds(0, 4)])
        v[...] = v[...] + 1.0
        pltpu.sync_copy(v.at[pl.ds(0, 4)], o1_hbm)


def kernel(a0, a1):
    return _sc_add_one(a0, a1)


# speedup vs baseline: 1.0600x; 1.0600x over previous
import functools

import jax
import jax.numpy as jnp
from jax import lax
from jax.experimental import pallas as pl
from jax.experimental.pallas import tpu as pltpu
from jax.experimental.pallas import tpu_sc as plsc


_MESH = plsc.VectorSubcoreMesh(
    core_axis_name="c", subcore_axis_name="s", num_cores=1, num_subcores=1
)


@functools.partial(
    pl.kernel,
    out_type=(
        jax.ShapeDtypeStruct((2,), jnp.float32),
        jax.ShapeDtypeStruct((4,), jnp.float32),
    ),
    mesh=_MESH,
    scratch_types=[pltpu.VMEM((16,), jnp.float32), pltpu.VMEM((16,), jnp.float32)],
)
def _sc_add_one(a0_hbm, a1_hbm, o0_hbm, o1_hbm, v0, v1):
    pltpu.sync_copy(a0_hbm, v0.at[pl.ds(0, 2)])
    pltpu.sync_copy(a1_hbm, v1.at[pl.ds(0, 4)])
    v0[...] = v0[...] + 1.0
    v1[...] = v1[...] + 1.0
    pltpu.sync_copy(v0.at[pl.ds(0, 2)], o0_hbm)
    pltpu.sync_copy(v1.at[pl.ds(0, 4)], o1_hbm)


def kernel(a0, a1):
    return _sc_add_one(a0, a1)


# speedup vs baseline: 10.4263x; 9.8356x over previous
import jax
import jax.numpy as jnp
from jax.experimental import pallas as pl
from jax.experimental.pallas import tpu as pltpu


def _add_one_body(a0_ref, a1_ref, o0_ref, o1_ref):
    for i in range(2):
        o0_ref[i] = a0_ref[i] + 1.0
    for i in range(4):
        o1_ref[i] = a1_ref[i] + 1.0


def kernel(a0, a1):
    return pl.pallas_call(
        _add_one_body,
        in_specs=[
            pl.BlockSpec(memory_space=pltpu.SMEM),
            pl.BlockSpec(memory_space=pltpu.SMEM),
        ],
        out_specs=(
            pl.BlockSpec(memory_space=pltpu.SMEM),
            pl.BlockSpec(memory_space=pltpu.SMEM),
        ),
        out_shape=(
            jax.ShapeDtypeStruct((2,), jnp.float32),
            jax.ShapeDtypeStruct((4,), jnp.float32),
        ),
    )(a0, a1)


# speedup vs baseline: 11.1302x; 1.0675x over previous
import jax
import jax.numpy as jnp
from jax.experimental import pallas as pl
from jax.experimental.pallas import tpu as pltpu


def _add_one_body(a0_ref, a1_ref, o0_ref, o1_ref):
    o0_ref[...] = a0_ref[...] + 1.0
    o1_ref[...] = a1_ref[...] + 1.0


def kernel(a0, a1):
    return pl.pallas_call(
        _add_one_body,
        out_shape=(
            jax.ShapeDtypeStruct((2,), jnp.float32),
            jax.ShapeDtypeStruct((4,), jnp.float32),
        ),
        compiler_params=pltpu.CompilerParams(
            disable_bounds_checks=True,
            disable_semaphore_checks=True,
            skip_device_barrier=True,
        ),
    )(a0, a1)
